# baseline (speedup 1.0000x reference)
import functools

import jax
import jax.numpy as jnp
from jax import lax
from jax.experimental import pallas as pl
from jax.experimental.pallas import tpu as pltpu
from jax.experimental.pallas import tpu_sc as plsc

_NC, _NS = 2, 16
_NW = _NC * _NS
_L = 16


@functools.partial(jax.jit, static_argnums=(2, 3, 4))
def _sc_pos_add(x_flat, pe_flat, B, S, D):
    RPW = S // _NW
    RB = 16
    n_blk = RPW // RB
    blk_words = RB * D

    mesh = plsc.VectorSubcoreMesh(
        core_axis_name="c", subcore_axis_name="s",
        num_cores=_NC, num_subcores=_NS)

    def body(x_hbm, pe_hbm, out_hbm, pe_v, x_v):
        wid = lax.axis_index("s") * _NC + lax.axis_index("c")
        base = wid * RPW

        def blk(i, carry):
            s0 = base + i * RB
            pltpu.sync_copy(pe_hbm.at[pl.ds(s0 * D, blk_words)], pe_v)

            def bat(b, carry2):
                w0 = (b * S + s0) * D
                pltpu.sync_copy(x_hbm.at[pl.ds(w0, blk_words)], x_v)

                def add_slice(j, carry3):
                    sl = pl.ds(j * _L, _L)
                    x_v[sl] = x_v[sl] + pe_v[sl]
                    return carry3

                lax.fori_loop(0, blk_words // _L, add_slice, 0, unroll=8)
                pltpu.sync_copy(x_v, out_hbm.at[pl.ds(w0, blk_words)])
                return carry2

            lax.fori_loop(0, B, bat, 0)
            return carry

        lax.fori_loop(0, n_blk, blk, 0)

    return pl.kernel(
        body,
        out_type=jax.ShapeDtypeStruct((B * S * D,), jnp.float32),
        mesh=mesh,
        scratch_types=[
            pltpu.VMEM((blk_words,), jnp.float32),
            pltpu.VMEM((blk_words,), jnp.float32),
        ],
    )(x_flat, pe_flat)


def kernel(x, pos_embedding):
    B, S, D = x.shape
    out = _sc_pos_add(x.reshape(-1), pos_embedding.reshape(-1), B, S, D)
    return out.reshape(B, S, D)


# speedup vs baseline: 1.1559x; 1.1559x over previous
import functools

import jax
import jax.numpy as jnp
from jax import lax
from jax.experimental import pallas as pl
from jax.experimental.pallas import tpu as pltpu
from jax.experimental.pallas import tpu_sc as plsc

_NC, _NS = 2, 16
_NW = _NC * _NS
_L = 16


@functools.partial(jax.jit, static_argnums=(2, 3, 4))
def _sc_pos_add(x_flat, pe_flat, B, S, D):
    RPW = S // _NW
    RB = 8
    NI = RPW // RB
    NBLK = NI * B
    BW = RB * D
    PEW = RPW * D

    mesh = plsc.VectorSubcoreMesh(
        core_axis_name="c", subcore_axis_name="s",
        num_cores=_NC, num_subcores=_NS)

    def body(x_hbm, pe_hbm, out_hbm, pe_v, in0, in1, out0, out1,
             sem_pe, sem_i0, sem_i1, sem_o0, sem_o1):
        wid = lax.axis_index("s") * _NC + lax.axis_index("c")
        base = wid * RPW
        ins = (in0, in1)
        outs = (out0, out1)
        sem_in = (sem_i0, sem_i1)
        sem_out = (sem_o0, sem_o1)

        def x_off(k):
            b, i = k // NI, k % NI
            return (b * S + base + i * RB) * D

        def start_in(k, j):
            pltpu.make_async_copy(
                x_hbm.at[pl.ds(x_off(k), BW)], ins[j], sem_in[j]).start()

        def wait_in(j):
            pltpu.make_async_copy(
                x_hbm.at[pl.ds(0, BW)], ins[j], sem_in[j]).wait()

        def start_out(k, j):
            pltpu.make_async_copy(
                outs[j], out_hbm.at[pl.ds(x_off(k), BW)], sem_out[j]).start()

        def wait_out(j):
            pltpu.make_async_copy(
                outs[j], out_hbm.at[pl.ds(0, BW)], sem_out[j]).wait()

        def compute(k, j):
            pe0 = (k % NI) * BW

            def add_slice(t, carry):
                sl = pl.ds(t * _L, _L)
                outs[j][sl] = ins[j][sl] + pe_v[pl.ds(pe0 + t * _L, _L)]
                return carry

            lax.fori_loop(0, BW // _L, add_slice, 0, unroll=8)

        pe_cp = pltpu.make_async_copy(
            pe_hbm.at[pl.ds(base * D, PEW)], pe_v, sem_pe)
        pe_cp.start()
        start_in(0, 0)
        start_in(1, 1)
        pe_cp.wait()

        for j in (0, 1):
            wait_in(j)
            compute(j, j)
            start_out(j, j)
            start_in(j + 2, j)

        def step(t, carry):
            for j in (0, 1):
                k = 2 + 2 * t + j
                wait_in(j)
                wait_out(j)
                compute(k, j)
                start_out(k, j)
                start_in(k + 2, j)
            return carry

        lax.fori_loop(0, (NBLK - 4) // 2, step, 0)

        for j in (0, 1):
            k = NBLK - 2 + j
            wait_in(j)
            wait_out(j)
            compute(k, j)
            start_out(k, j)
        wait_out(0)
        wait_out(1)

    return pl.kernel(
        body,
        out_type=jax.ShapeDtypeStruct((B * S * D,), jnp.float32),
        mesh=mesh,
        scratch_types=[
            pltpu.VMEM((PEW,), jnp.float32),
            pltpu.VMEM((BW,), jnp.float32),
            pltpu.VMEM((BW,), jnp.float32),
            pltpu.VMEM((BW,), jnp.float32),
            pltpu.VMEM((BW,), jnp.float32),
            pltpu.SemaphoreType.DMA,
            pltpu.SemaphoreType.DMA,
            pltpu.SemaphoreType.DMA,
            pltpu.SemaphoreType.DMA,
            pltpu.SemaphoreType.DMA,
        ],
    )(x_flat, pe_flat)


def kernel(x, pos_embedding):
    B, S, D = x.shape
    out = _sc_pos_add(x.reshape(-1), pos_embedding.reshape(-1), B, S, D)
    return out.reshape(B, S, D)


# speedup vs baseline: 4.0844x; 3.5334x over previous
import functools

import jax
import jax.numpy as jnp
from jax import lax
from jax.experimental import pallas as pl
from jax.experimental.pallas import tpu as pltpu
from jax.experimental.pallas import tpu_sc as plsc

_NC, _NS = 2, 16
_NW = _NC * _NS
_L = 16


@functools.partial(jax.jit, static_argnums=(2, 3, 4))
def _sc_pos_add(x2, pe, B, S, D):
    RPW = S // _NW
    RB = 16
    NI = RPW // RB
    NBLK = NI * B
    NCOL = D // _L

    mesh = plsc.VectorSubcoreMesh(
        core_axis_name="c", subcore_axis_name="s",
        num_cores=_NC, num_subcores=_NS)

    def body(x_hbm, pe_hbm, out_hbm, pe0, pe1, in0, in1, out0, out1,
             sem_p0, sem_p1, sem_i0, sem_i1, sem_o0, sem_o1):
        wid = lax.axis_index("s") * _NC + lax.axis_index("c")
        base = wid * RPW
        pes, sem_pe = (pe0, pe1), (sem_p0, sem_p1)
        ins, sem_in = (in0, in1), (sem_i0, sem_i1)
        outs, sem_out = (out0, out1), (sem_o0, sem_o1)

        def x_row(k):
            return (k % B) * S + base + (k // B) * RB

        def start_pe(i, p):
            pltpu.make_async_copy(
                pe_hbm.at[pl.ds(base + i * RB, RB)], pes[p], sem_pe[p]).start()

        def wait_pe(p):
            pltpu.make_async_copy(
                pe_hbm.at[pl.ds(0, RB)], pes[p], sem_pe[p]).wait()

        def start_in(k, j):
            pltpu.make_async_copy(
                x_hbm.at[pl.ds(x_row(k), RB)], ins[j], sem_in[j]).start()

        def wait_in(j):
            pltpu.make_async_copy(
                x_hbm.at[pl.ds(0, RB)], ins[j], sem_in[j]).wait()

        def start_out(k, j):
            pltpu.make_async_copy(
                outs[j], out_hbm.at[pl.ds(x_row(k), RB)], sem_out[j]).start()

        def wait_out(j):
            pltpu.make_async_copy(
                outs[j], out_hbm.at[pl.ds(0, RB)], sem_out[j]).wait()

        def compute(j, p):
            @plsc.parallel_loop(0, RB * NCOL, unroll=8)
            def _(t):
                r = t // NCOL
                sl = pl.ds((t % NCOL) * _L, _L)
                outs[j][r, sl] = ins[j][r, sl] + pes[p][r, sl]

        start_pe(0, 0)
        start_pe(1, 1)
        start_in(0, 0)
        start_in(1, 1)

        for k in range(NBLK):
            j, i, p = k % 2, k // B, (k // B) % 2
            if k == B:
                start_pe(2, 0)
            if k == 2 * B:
                start_pe(3, 1)
            if k % B == 0:
                wait_pe(p)
            wait_in(j)
            if k >= 2:
                wait_out(j)
            compute(j, p)
            start_out(k, j)
            if k + 2 < NBLK:
                start_in(k + 2, j)
        wait_out(0)
        wait_out(1)

    return pl.kernel(
        body,
        out_type=jax.ShapeDtypeStruct((B * S, D), jnp.float32),
        mesh=mesh,
        scratch_types=[
            pltpu.VMEM((RB, D), jnp.float32),
            pltpu.VMEM((RB, D), jnp.float32),
            pltpu.VMEM((RB, D), jnp.float32),
            pltpu.VMEM((RB, D), jnp.float32),
            pltpu.VMEM((RB, D), jnp.float32),
            pltpu.VMEM((RB, D), jnp.float32),
            pltpu.SemaphoreType.DMA,
            pltpu.SemaphoreType.DMA,
            pltpu.SemaphoreType.DMA,
            pltpu.SemaphoreType.DMA,
            pltpu.SemaphoreType.DMA,
            pltpu.SemaphoreType.DMA,
        ],
    )(x2, pe)


def kernel(x, pos_embedding):
    B, S, D = x.shape
    out = _sc_pos_add(x.reshape(B * S, D), pos_embedding, B, S, D)
    return out.reshape(B, S, D)
